# Initial kernel scaffold; baseline (speedup 1.0000x reference)
import jax
import jax.numpy as jnp
from jax.experimental import pallas as pl


def kernel(loc_preds, loc_targets, cls_preds, cls_targets):
    raise NotImplementedError("write your pallas kernel here")


# speedup vs baseline: 1.9236x; 1.9236x over previous
import jax
import jax.numpy as jnp
from jax import lax
from jax.experimental import pallas as pl
from jax.experimental.pallas import tpu as pltpu, tpu_sc as plsc

NC, NS, L = 2, 16, 16
NW = NC * NS
B, A, C = 32, 65536, 8
TOT = B * A
APW = TOT // NW
CH = 4096
NCHUNK = APW // CH
CE = CH * C


def _sc_body(lp_hbm, lt_hbm, ct_hbm, out_hbm, lp_buf, lt_buf, ct_buf, res_buf):
    wid = lax.axis_index("s") * NC + lax.axis_index("c")
    abase = wid * APW
    iota = lax.iota(jnp.int32, L)
    low_half = iota < 8

    def chunk_body(c, carry):
        acc, cnt = carry
        astart = abase + c * CH
        pltpu.sync_copy(lp_hbm.at[pl.ds(astart * C, CE)], lp_buf)
        pltpu.sync_copy(lt_hbm.at[pl.ds(astart * C, CE)], lt_buf)
        pltpu.sync_copy(ct_hbm.at[pl.ds(astart, CH)], ct_buf)

        def group_body(g, carry):
            acc, cnt = carry
            tg = ct_buf[pl.ds(g * L, L)]
            cnt = cnt + jnp.where(tg > 0, 1.0, 0.0).astype(jnp.float32)
            ebase = g * (L * C)
            for k in range(8):
                a = lp_buf[pl.ds(ebase + k * L, L)]
                b = lt_buf[pl.ds(ebase + k * L, L)]
                t0 = tg[2 * k]
                t1 = tg[2 * k + 1]
                tv = jnp.where(low_half, t0, t1)
                d = jnp.where(tv > 0, a - b, 0.0)
                absd = jnp.abs(d)
                acc = acc + jnp.where(absd < 1.0, (0.5 * d) * d, absd - 0.5)
            return acc, cnt

        return lax.fori_loop(0, CH // L, group_body, (acc, cnt))

    zeros = jnp.zeros((L,), jnp.float32)
    acc, cnt = lax.fori_loop(0, NCHUNK, chunk_body, (zeros, zeros))
    res_buf[pl.ds(0, L)] = acc
    res_buf[pl.ds(L, L)] = cnt
    pltpu.sync_copy(res_buf, out_hbm.at[wid])


def kernel(loc_preds, loc_targets, cls_preds, cls_targets):
    lp = loc_preds.reshape(-1)
    lt = loc_targets.reshape(-1)
    ct = cls_targets.astype(jnp.int32).reshape(-1)
    mesh = plsc.VectorSubcoreMesh(
        core_axis_name="c", subcore_axis_name="s",
        num_cores=NC, num_subcores=NS)
    out = pl.kernel(
        _sc_body,
        out_type=jax.ShapeDtypeStruct((NW, 2 * L), jnp.float32),
        mesh=mesh,
        scratch_types=[
            pltpu.VMEM((CE,), jnp.float32),
            pltpu.VMEM((CE,), jnp.float32),
            pltpu.VMEM((CH,), jnp.int32),
            pltpu.VMEM((2 * L,), jnp.float32),
        ],
    )(lp, lt, ct)
    sl1_sum = jnp.sum(out[:, :L])
    n = jnp.sum(out[:, L:])
    return 0.2 * (sl1_sum / n)


# speedup vs baseline: 12.3829x; 6.4374x over previous
import jax
import jax.numpy as jnp
from jax import lax
from jax.experimental import pallas as pl
from jax.experimental.pallas import tpu as pltpu, tpu_sc as plsc

NC, NS, L = 2, 16, 16
NW = NC * NS
B, A, C = 32, 65536, 8
KT = A // 128
CK = 32
NCHUNK = KT // CK
ROWS = CK * C


def _sc_body(lp_hbm, lt_hbm, ct_hbm, out_hbm, lp_buf, lt_buf, ct_buf, res_buf):
    b = lax.axis_index("s") * NC + lax.axis_index("c")
    r = b // 8
    i = b % 8

    def chunk_body(c0, carry):
        accq, acca, acct, cnt = carry
        pltpu.sync_copy(lp_hbm.at[b, pl.ds(c0 * ROWS, ROWS), :], lp_buf)
        pltpu.sync_copy(lt_hbm.at[b, pl.ds(c0 * ROWS, ROWS), :], lt_buf)
        pltpu.sync_copy(ct_hbm.at[r, pl.ds(c0 * CK, CK), i, :], ct_buf)

        def tile_body(kk, carry):
            accq, acca, acct, cnt = carry
            m = []
            for l in range(8):
                tl = ct_buf[kk, pl.ds(l * L, L)]
                ml = jnp.where(tl > 0, 1.0, 0.0).astype(jnp.float32)
                cnt = cnt + ml
                m.append(ml)
            for c in range(8):
                row = kk * 8 + c
                for l in range(8):
                    a = lp_buf[row, pl.ds(l * L, L)]
                    bb = lt_buf[row, pl.ds(l * L, L)]
                    d = (a - bb) * m[l]
                    absd = jnp.abs(d)
                    t = jnp.minimum(absd, 1.0)
                    acca = acca + absd
                    acct = acct + t
                    accq = accq + (0.5 * t) * t
            return accq, acca, acct, cnt

        return lax.fori_loop(0, CK, tile_body, (accq, acca, acct, cnt))

    z = jnp.zeros((L,), jnp.float32)
    accq, acca, acct, cnt = lax.fori_loop(
        0, NCHUNK, chunk_body, (z, z, z, z))
    res_buf[pl.ds(0, L)] = accq + acca - acct
    res_buf[pl.ds(L, L)] = cnt
    pltpu.sync_copy(res_buf, out_hbm.at[b])


def kernel(loc_preds, loc_targets, cls_preds, cls_targets):
    lp = loc_preds.reshape(B, KT, 128, C).transpose(0, 1, 3, 2).reshape(B, KT * C, 128)
    lt = loc_targets.reshape(B, KT, 128, C).transpose(0, 1, 3, 2).reshape(B, KT * C, 128)
    ct = cls_targets.astype(jnp.int32).reshape(B // 8, 8, KT, 128).transpose(0, 2, 1, 3)
    mesh = plsc.VectorSubcoreMesh(
        core_axis_name="c", subcore_axis_name="s",
        num_cores=NC, num_subcores=NS)
    out = pl.kernel(
        _sc_body,
        out_type=jax.ShapeDtypeStruct((NW, 2 * L), jnp.float32),
        mesh=mesh,
        scratch_types=[
            pltpu.VMEM((ROWS, 128), jnp.float32),
            pltpu.VMEM((ROWS, 128), jnp.float32),
            pltpu.VMEM((CK, 128), jnp.int32),
            pltpu.VMEM((2 * L,), jnp.float32),
        ],
    )(lp, lt, ct)
    sl1_sum = jnp.sum(out[:, :L])
    n = jnp.sum(out[:, L:])
    return 0.2 * (sl1_sum / n)
